# baseline (speedup 1.0000x reference)
import functools

import jax
import jax.numpy as jnp
from jax import lax
from jax.experimental import pallas as pl
from jax.experimental.pallas import tpu as pltpu
from jax.experimental.pallas import tpu_sc as plsc

N = 10000
NP = 10240
E = 320000
EP = 327680
D = 128
CH = 128
NW = 32
EPW = EP // NW
NCHUNK = EPW // CH
RPT = NP // 16
LN_EPS = 1e-5
DH = 64

_mesh = plsc.VectorSubcoreMesh(core_axis_name="c", subcore_axis_name="s")


NBUF = 4
_DNCH = (EP // CH) // 32
_DNGRP = _DNCH // NBUF
_DPLAST = (_DNGRP - 1) % 2


@functools.partial(
    pl.kernel,
    out_type=jax.ShapeDtypeStruct((2 * NP, 8), jnp.float32),
    mesh=_mesh,
    scratch_types=[
        pltpu.VMEM((2, NBUF, CH), jnp.int32),
        pltpu.VMEM((CH, 8), jnp.float32),
        pltpu.VMEM_SHARED((NP, 8), jnp.float32),
        pltpu.SemaphoreType.DMA((2,)),
        pltpu.SemaphoreType.DMA((NBUF,)),
    ],
    compiler_params=pltpu.CompilerParams(use_tc_tiling_on_sc=False),
)
def _deg_kernel(dst_hbm, ones_hbm, zeros_hbm, out_hbm,
                didx, ones_v, acc_sh, isem, ssem):
    c = lax.axis_index("c")
    s = lax.axis_index("s")
    doff = (s * 2 + c) * _DNCH

    def load_idx(g, ph):
        pltpu.async_copy(dst_hbm.at[pl.ds(doff + g * NBUF, NBUF)],
                         didx.at[ph], isem.at[ph])

    def wait_idx(ph):
        pltpu.make_async_copy(dst_hbm.at[pl.ds(doff, NBUF)],
                              didx.at[ph], isem.at[ph]).wait()

    pltpu.sync_copy(zeros_hbm, acc_sh.at[pl.ds(s * RPT, RPT)])
    pltpu.sync_copy(ones_hbm, ones_v)
    load_idx(0, 0)
    wait_idx(0)
    load_idx(1, 1)
    plsc.subcore_barrier()

    def group(g, carry):
        p = lax.rem(g, 2)
        q = 1 - p
        for b in range(NBUF):
            pltpu.async_copy(ones_v, acc_sh.at[didx.at[p, b]],
                             ssem.at[b], add=True)
        wait_idx(q)
        for b in range(NBUF):
            pltpu.make_async_copy(ones_v, acc_sh.at[didx.at[p, b]],
                                  ssem.at[b]).wait()

        @pl.when(g < _DNGRP - 2)
        def _():
            load_idx(g + 2, p)

        return carry

    lax.fori_loop(0, _DNGRP - 1, group, 0)
    for b in range(NBUF):
        pltpu.async_copy(ones_v, acc_sh.at[didx.at[_DPLAST, b]],
                         ssem.at[b], add=True)
    for b in range(NBUF):
        pltpu.make_async_copy(ones_v, acc_sh.at[didx.at[_DPLAST, b]],
                              ssem.at[b]).wait()

    plsc.subcore_barrier()
    out_off = c * NP + s * RPT
    pltpu.sync_copy(acc_sh.at[pl.ds(s * RPT, RPT)],
                    out_hbm.at[pl.ds(out_off, RPT)])


def _make_agg(edge_split):
  ncht = ((EP // CH) // 32) if edge_split else ((EP // CH) // 16)
  ngrp = ncht // NBUF
  p_last = (ngrp - 1) % 2

  @functools.partial(
      pl.kernel,
      out_type=jax.ShapeDtypeStruct((2 * NP, DH), jnp.float32),
      mesh=_mesh,
      scratch_types=[
          pltpu.VMEM((2, NBUF, CH), jnp.int32),
          pltpu.VMEM((2, NBUF, CH), jnp.int32),
          pltpu.VMEM((NBUF, CH, DH), jnp.float32),
          pltpu.VMEM_SHARED((NP, DH), jnp.float32),
          pltpu.SemaphoreType.DMA((2,)),
          pltpu.SemaphoreType.DMA((NBUF,)),
          pltpu.SemaphoreType.DMA((NBUF,)),
      ],
      compiler_params=pltpu.CompilerParams(use_tc_tiling_on_sc=False),
  )
  def _agg_kernel(table_hbm, srcx_hbm, dst_hbm, zeros_hbm, out_hbm,
                  sidx, didx, rows_v, acc_sh, isem, gsem, ssem):
    c = lax.axis_index("c")
    s = lax.axis_index("s")
    pltpu.sync_copy(zeros_hbm, acc_sh.at[pl.ds(s * RPT, RPT)])
    if edge_split:
        soff = (s * 2 + c) * ncht
        doff = soff
    else:
        soff = c * (EP // CH) + s * ncht
        doff = s * ncht

    def load_idx(g, ph):
        pltpu.async_copy(srcx_hbm.at[pl.ds(soff + g * NBUF, NBUF)],
                         sidx.at[ph], isem.at[ph])
        pltpu.async_copy(dst_hbm.at[pl.ds(doff + g * NBUF, NBUF)],
                         didx.at[ph], isem.at[ph])

    def wait_idx(ph):
        pltpu.make_async_copy(srcx_hbm.at[pl.ds(soff, NBUF)],
                              sidx.at[ph], isem.at[ph]).wait()
        pltpu.make_async_copy(dst_hbm.at[pl.ds(doff, NBUF)],
                              didx.at[ph], isem.at[ph]).wait()

    load_idx(0, 0)
    wait_idx(0)
    load_idx(1, 1)
    plsc.subcore_barrier()

    for b in range(NBUF):
        pltpu.async_copy(table_hbm.at[sidx.at[0, b]], rows_v.at[b], gsem.at[b])

    def group(g, carry):
        p = lax.rem(g, 2)
        q = 1 - p
        for b in range(NBUF):
            pltpu.make_async_copy(table_hbm.at[sidx.at[p, b]], rows_v.at[b],
                                  gsem.at[b]).wait()
            pltpu.async_copy(rows_v.at[b], acc_sh.at[didx.at[p, b]],
                             ssem.at[b], add=True)
        wait_idx(q)
        for b in range(NBUF):
            pltpu.make_async_copy(rows_v.at[b], acc_sh.at[didx.at[p, b]],
                                  ssem.at[b]).wait()
            pltpu.async_copy(table_hbm.at[sidx.at[q, b]], rows_v.at[b],
                             gsem.at[b])

        @pl.when(g < ngrp - 2)
        def _():
            load_idx(g + 2, p)

        return carry

    lax.fori_loop(0, ngrp - 1, group, 0)
    for b in range(NBUF):
        pltpu.make_async_copy(table_hbm.at[sidx.at[p_last, b]], rows_v.at[b],
                              gsem.at[b]).wait()
        pltpu.async_copy(rows_v.at[b], acc_sh.at[didx.at[p_last, b]],
                         ssem.at[b], add=True)
    for b in range(NBUF):
        pltpu.make_async_copy(rows_v.at[b], acc_sh.at[didx.at[p_last, b]],
                              ssem.at[b]).wait()

    plsc.subcore_barrier()
    out_off = c * NP + s * RPT
    pltpu.sync_copy(acc_sh.at[pl.ds(s * RPT, RPT)],
                    out_hbm.at[pl.ds(out_off, RPT)])

  return _agg_kernel


_agg64 = _make_agg(edge_split=False)
_agg3 = _make_agg(edge_split=True)


BM = 1024
GRID = NP // BM


def _dinv_block(degp_ref):
    deg = degp_ref[0, :, 0:1] + degp_ref[1, :, 0:1] + 1.0
    return lax.rsqrt(deg)


def _split_store(out_ref, res, dh):
    out_ref[0] = res[:, :dh]
    out_ref[1] = res[:, dh:]


def _merge(ref2):
    return jnp.concatenate([ref2[0], ref2[1]], axis=1)


def _prep_body(degp_ref, x_ref, w_ref, out_ref):
    dinv = _dinv_block(degp_ref)
    h = jnp.dot(x_ref[...], w_ref[...], preferred_element_type=jnp.float32)
    _split_store(out_ref, h * dinv, DH)


def _mid_body(dh_out, agg_ref, hp_ref, degp_ref, b_ref, g_ref, be_ref, w_ref,
              out_ref):
    dinv = _dinv_block(degp_ref)
    t = _merge(agg_ref) + _merge(hp_ref)
    z = t * dinv + b_ref[...]
    mu = jnp.mean(z, axis=1, keepdims=True)
    zc = z - mu
    var = jnp.mean(zc * zc, axis=1, keepdims=True)
    zn = zc * lax.rsqrt(var + LN_EPS) * g_ref[...] + be_ref[...]
    r = jnp.maximum(zn, 0.0)
    res = jnp.dot(r, w_ref[...], preferred_element_type=jnp.float32) * dinv
    _split_store(out_ref, res, dh_out)


def _final_body(agg_ref, hp_ref, degp_ref, b_ref, out_ref):
    dinv = _dinv_block(degp_ref)
    t64 = agg_ref[0] + agg_ref[1] + hp_ref[0]
    t = jnp.concatenate([t64, jnp.zeros_like(t64)], axis=1)
    z = t * dinv + b_ref[...]
    col = lax.broadcasted_iota(jnp.int32, z.shape, 1)
    valid = col < 40
    zm = jnp.where(valid, z, -jnp.inf)
    m = jnp.max(zm, axis=1, keepdims=True)
    e = jnp.where(valid, jnp.exp(z - m), 0.0)
    lse = jnp.log(jnp.sum(e, axis=1, keepdims=True))
    out_ref[...] = z - m - lse


_DEGP_SPEC = pl.BlockSpec((2, BM, 8), lambda i: (0, i, 0))
_ROWS_SPEC = pl.BlockSpec((BM, D), lambda i: (i, 0))
_SPLIT_SPEC = pl.BlockSpec((2, BM, DH), lambda i: (0, i, 0))
_HALF_SPEC = pl.BlockSpec((1, BM, DH), lambda i: (0, i, 0))
_W_SPEC = pl.BlockSpec((D, D), lambda i: (0, 0))
_VEC_SPEC = pl.BlockSpec((1, D), lambda i: (0, 0))
_OUT_SDS = jax.ShapeDtypeStruct((NP, D), jnp.float32)
_SPLIT_SDS = jax.ShapeDtypeStruct((2, NP, DH), jnp.float32)


def _prep_call(degp, x_p, w):
    return pl.pallas_call(
        _prep_body, grid=(GRID,),
        in_specs=[_DEGP_SPEC, _ROWS_SPEC, _W_SPEC],
        out_specs=_SPLIT_SPEC, out_shape=_SPLIT_SDS,
    )(degp, x_p, w)


def _mid_call(agg, hp, degp, b, g, be, w):
    return pl.pallas_call(
        functools.partial(_mid_body, DH), grid=(GRID,),
        in_specs=[_SPLIT_SPEC, _SPLIT_SPEC, _DEGP_SPEC,
                  _VEC_SPEC, _VEC_SPEC, _VEC_SPEC, _W_SPEC],
        out_specs=_SPLIT_SPEC, out_shape=_SPLIT_SDS,
    )(agg, hp, degp, b, g, be, w)


def _final_call(agg, hp, degp, b):
    return pl.pallas_call(
        _final_body, grid=(GRID,),
        in_specs=[_SPLIT_SPEC, _HALF_SPEC, _DEGP_SPEC, _VEC_SPEC],
        out_specs=_ROWS_SPEC, out_shape=_OUT_SDS,
    )(agg, hp, degp, b)


def kernel(x, adj_t, W1, b1, g1, be1, W2, b2, g2, be2, W3, b3):
    src = adj_t[0].astype(jnp.int32)
    dst = adj_t[1].astype(jnp.int32)
    pad_idx = jnp.full((EP - E,), N, jnp.int32)
    src_p = jnp.concatenate([src, pad_idx])
    dst_p = jnp.concatenate([dst, pad_idx])
    src2d = src_p.reshape(EP // CH, CH)
    dst2d = dst_p.reshape(EP // CH, CH)
    srcx = jnp.concatenate([src2d, src2d + NP]).astype(jnp.int32)
    x_p = jnp.pad(x, ((0, NP - N), (0, 0)))
    ones8 = jnp.ones((CH, 8), jnp.float32)
    zeros8 = jnp.zeros((RPT, 8), jnp.float32)
    zerosH = jnp.zeros((RPT, DH), jnp.float32)
    W3p = jnp.pad(W3, ((0, 0), (0, D - 40)))
    b1r = b1.reshape(1, D)
    g1r = g1.reshape(1, D)
    be1r = be1.reshape(1, D)
    b2r = b2.reshape(1, D)
    g2r = g2.reshape(1, D)
    be2r = be2.reshape(1, D)
    b3r = jnp.pad(b3, (0, D - 40)).reshape(1, D)

    degp = _deg_kernel(dst2d, ones8, zeros8).reshape(2, NP, 8)
    h1 = _prep_call(degp, x_p, W1)
    agg1 = _agg64(h1.reshape(2 * NP, DH), srcx, dst2d, zerosH)
    h2 = _mid_call(agg1.reshape(2, NP, DH), h1, degp, b1r, g1r, be1r, W2)
    agg2 = _agg64(h2.reshape(2 * NP, DH), srcx, dst2d, zerosH)
    h3 = _mid_call(agg2.reshape(2, NP, DH), h2, degp, b2r, g2r, be2r, W3p)
    agg3 = _agg3(h3[0], src2d, dst2d, zerosH)
    out = _final_call(agg3.reshape(2, NP, DH), h3[0:1], degp, b3r)
    return out[:N, :40]


# speedup vs baseline: 1.1853x; 1.1853x over previous
import functools

import jax
import jax.numpy as jnp
from jax import lax
from jax.experimental import pallas as pl
from jax.experimental.pallas import tpu as pltpu
from jax.experimental.pallas import tpu_sc as plsc

N = 10000
NP = 10240
E = 320000
EP = 327680
D = 128
CH = 128
NW = 32
EPW = EP // NW
NCHUNK = EPW // CH
RPT = NP // 16
LN_EPS = 1e-5
DH = 64

_mesh = plsc.VectorSubcoreMesh(core_axis_name="c", subcore_axis_name="s")


NBUF = 4
_DNCH = (EP // CH) // 32
_DNGRP = _DNCH // NBUF
_DPLAST = (_DNGRP - 1) % 2


@functools.partial(
    pl.kernel,
    out_type=jax.ShapeDtypeStruct((2 * NP, 8), jnp.float32),
    mesh=_mesh,
    scratch_types=[
        pltpu.VMEM((2, NBUF, CH), jnp.int32),
        pltpu.VMEM((CH, 8), jnp.float32),
        pltpu.VMEM_SHARED((NP, 8), jnp.float32),
        pltpu.SemaphoreType.DMA((2,)),
        pltpu.SemaphoreType.DMA((NBUF,)),
    ],
    compiler_params=pltpu.CompilerParams(use_tc_tiling_on_sc=False),
)
def _deg_kernel(dst_hbm, ones_hbm, zeros_hbm, out_hbm,
                didx, ones_v, acc_sh, isem, ssem):
    c = lax.axis_index("c")
    s = lax.axis_index("s")
    doff = (s * 2 + c) * _DNCH

    def load_idx(g, ph):
        pltpu.async_copy(dst_hbm.at[pl.ds(doff + g * NBUF, NBUF)],
                         didx.at[ph], isem.at[ph])

    def wait_idx(ph):
        pltpu.make_async_copy(dst_hbm.at[pl.ds(doff, NBUF)],
                              didx.at[ph], isem.at[ph]).wait()

    pltpu.sync_copy(zeros_hbm, acc_sh.at[pl.ds(s * RPT, RPT)])
    pltpu.sync_copy(ones_hbm, ones_v)
    load_idx(0, 0)
    wait_idx(0)
    load_idx(1, 1)
    plsc.subcore_barrier()

    def group(g, carry):
        p = lax.rem(g, 2)
        q = 1 - p
        for b in range(NBUF):
            pltpu.async_copy(ones_v, acc_sh.at[didx.at[p, b]],
                             ssem.at[b], add=True)
        wait_idx(q)
        for b in range(NBUF):
            pltpu.make_async_copy(ones_v, acc_sh.at[didx.at[p, b]],
                                  ssem.at[b]).wait()

        @pl.when(g < _DNGRP - 2)
        def _():
            load_idx(g + 2, p)

        return carry

    lax.fori_loop(0, _DNGRP - 1, group, 0)
    for b in range(NBUF):
        pltpu.async_copy(ones_v, acc_sh.at[didx.at[_DPLAST, b]],
                         ssem.at[b], add=True)
    for b in range(NBUF):
        pltpu.make_async_copy(ones_v, acc_sh.at[didx.at[_DPLAST, b]],
                              ssem.at[b]).wait()

    plsc.subcore_barrier()
    out_off = c * NP + s * RPT
    pltpu.sync_copy(acc_sh.at[pl.ds(s * RPT, RPT)],
                    out_hbm.at[pl.ds(out_off, RPT)])


def _make_agg(edge_split, dh=DH):
  ncht = ((EP // CH) // 32) if edge_split else ((EP // CH) // 16)
  ngrp = ncht // NBUF
  p_last = (ngrp - 1) % 2

  @functools.partial(
      pl.kernel,
      out_type=jax.ShapeDtypeStruct((2 * NP, dh), jnp.float32),
      mesh=_mesh,
      scratch_types=[
          pltpu.VMEM((2, NBUF, CH), jnp.int32),
          pltpu.VMEM((2, NBUF, CH), jnp.int32),
          pltpu.VMEM((NBUF, CH, dh), jnp.float32),
          pltpu.VMEM_SHARED((NP, dh), jnp.float32),
          pltpu.SemaphoreType.DMA((2,)),
          pltpu.SemaphoreType.DMA((NBUF,)),
          pltpu.SemaphoreType.DMA((NBUF,)),
      ],
      compiler_params=pltpu.CompilerParams(use_tc_tiling_on_sc=False),
  )
  def _agg_kernel(table_hbm, srcx_hbm, dst_hbm, zeros_hbm, out_hbm,
                  sidx, didx, rows_v, acc_sh, isem, gsem, ssem):
    c = lax.axis_index("c")
    s = lax.axis_index("s")
    pltpu.sync_copy(zeros_hbm, acc_sh.at[pl.ds(s * RPT, RPT)])
    if edge_split:
        soff = (s * 2 + c) * ncht
        doff = soff
    else:
        soff = c * (EP // CH) + s * ncht
        doff = s * ncht

    def load_idx(g, ph):
        pltpu.async_copy(srcx_hbm.at[pl.ds(soff + g * NBUF, NBUF)],
                         sidx.at[ph], isem.at[ph])
        pltpu.async_copy(dst_hbm.at[pl.ds(doff + g * NBUF, NBUF)],
                         didx.at[ph], isem.at[ph])

    def wait_idx(ph):
        pltpu.make_async_copy(srcx_hbm.at[pl.ds(soff, NBUF)],
                              sidx.at[ph], isem.at[ph]).wait()
        pltpu.make_async_copy(dst_hbm.at[pl.ds(doff, NBUF)],
                              didx.at[ph], isem.at[ph]).wait()

    load_idx(0, 0)
    wait_idx(0)
    load_idx(1, 1)
    plsc.subcore_barrier()

    for b in range(NBUF):
        pltpu.async_copy(table_hbm.at[sidx.at[0, b]], rows_v.at[b], gsem.at[b])

    def group(g, carry):
        p = lax.rem(g, 2)
        q = 1 - p
        for b in range(NBUF):
            pltpu.make_async_copy(table_hbm.at[sidx.at[p, b]], rows_v.at[b],
                                  gsem.at[b]).wait()
            pltpu.async_copy(rows_v.at[b], acc_sh.at[didx.at[p, b]],
                             ssem.at[b], add=True)
        wait_idx(q)
        for b in range(NBUF):
            pltpu.make_async_copy(rows_v.at[b], acc_sh.at[didx.at[p, b]],
                                  ssem.at[b]).wait()
            pltpu.async_copy(table_hbm.at[sidx.at[q, b]], rows_v.at[b],
                             gsem.at[b])

        @pl.when(g < ngrp - 2)
        def _():
            load_idx(g + 2, p)

        return carry

    lax.fori_loop(0, ngrp - 1, group, 0)
    for b in range(NBUF):
        pltpu.make_async_copy(table_hbm.at[sidx.at[p_last, b]], rows_v.at[b],
                              gsem.at[b]).wait()
        pltpu.async_copy(rows_v.at[b], acc_sh.at[didx.at[p_last, b]],
                         ssem.at[b], add=True)
    for b in range(NBUF):
        pltpu.make_async_copy(rows_v.at[b], acc_sh.at[didx.at[p_last, b]],
                              ssem.at[b]).wait()

    plsc.subcore_barrier()
    out_off = c * NP + s * RPT
    pltpu.sync_copy(acc_sh.at[pl.ds(s * RPT, RPT)],
                    out_hbm.at[pl.ds(out_off, RPT)])

  return _agg_kernel


_agg64 = _make_agg(edge_split=False)
_agg32 = _make_agg(edge_split=False, dh=32)


BM = 1024
GRID = NP // BM


def _dinv_block(degp_ref):
    deg = degp_ref[0, :, 0:1] + degp_ref[1, :, 0:1] + 1.0
    return lax.rsqrt(deg)


def _split_store(out_ref, res, dh):
    out_ref[0] = res[:, :dh]
    out_ref[1] = res[:, dh:]


def _merge(ref2):
    return jnp.concatenate([ref2[0], ref2[1]], axis=1)


def _prep_body(degp_ref, x_ref, w_ref, out_ref):
    dinv = _dinv_block(degp_ref)
    h = jnp.dot(x_ref[...], w_ref[...], preferred_element_type=jnp.float32)
    _split_store(out_ref, h * dinv, DH)


def _mid_body(dh_out, agg_ref, hp_ref, degp_ref, b_ref, g_ref, be_ref, w_ref,
              out_ref):
    dinv = _dinv_block(degp_ref)
    t = _merge(agg_ref) + _merge(hp_ref)
    z = t * dinv + b_ref[...]
    mu = jnp.mean(z, axis=1, keepdims=True)
    zc = z - mu
    var = jnp.mean(zc * zc, axis=1, keepdims=True)
    zn = zc * lax.rsqrt(var + LN_EPS) * g_ref[...] + be_ref[...]
    r = jnp.maximum(zn, 0.0)
    res = jnp.dot(r, w_ref[...], preferred_element_type=jnp.float32) * dinv
    _split_store(out_ref, res, dh_out)


def _final_body(agg_ref, hp_ref, degp_ref, b_ref, out_ref):
    dinv = _dinv_block(degp_ref)
    t = _merge(agg_ref) + _merge(hp_ref)
    z = t * dinv + b_ref[...]
    col = lax.broadcasted_iota(jnp.int32, z.shape, 1)
    valid = col < 40
    zm = jnp.where(valid, z, -jnp.inf)
    m = jnp.max(zm, axis=1, keepdims=True)
    e = jnp.where(valid, jnp.exp(z - m), 0.0)
    lse = jnp.log(jnp.sum(e, axis=1, keepdims=True))
    out_ref[...] = z - m - lse


_DEGP_SPEC = pl.BlockSpec((2, BM, 8), lambda i: (0, i, 0))
_ROWS_SPEC = pl.BlockSpec((BM, D), lambda i: (i, 0))
_SPLIT_SPEC = pl.BlockSpec((2, BM, DH), lambda i: (0, i, 0))
_HALF_SPEC = pl.BlockSpec((1, BM, DH), lambda i: (0, i, 0))
_SPLIT32_SPEC = pl.BlockSpec((2, BM, 32), lambda i: (0, i, 0))
_SPLIT32_SDS = jax.ShapeDtypeStruct((2, NP, 32), jnp.float32)
_W64_SPEC = pl.BlockSpec((D, 64), lambda i: (0, 0))
_VEC64_SPEC = pl.BlockSpec((1, 64), lambda i: (0, 0))
_OUT64_SPEC = pl.BlockSpec((BM, 64), lambda i: (i, 0))
_OUT64_SDS = jax.ShapeDtypeStruct((NP, 64), jnp.float32)
_W_SPEC = pl.BlockSpec((D, D), lambda i: (0, 0))
_VEC_SPEC = pl.BlockSpec((1, D), lambda i: (0, 0))
_OUT_SDS = jax.ShapeDtypeStruct((NP, D), jnp.float32)
_SPLIT_SDS = jax.ShapeDtypeStruct((2, NP, DH), jnp.float32)


def _prep_call(degp, x_p, w):
    return pl.pallas_call(
        _prep_body, grid=(GRID,),
        in_specs=[_DEGP_SPEC, _ROWS_SPEC, _W_SPEC],
        out_specs=_SPLIT_SPEC, out_shape=_SPLIT_SDS,
    )(degp, x_p, w)


def _mid_call(agg, hp, degp, b, g, be, w):
    return pl.pallas_call(
        functools.partial(_mid_body, DH), grid=(GRID,),
        in_specs=[_SPLIT_SPEC, _SPLIT_SPEC, _DEGP_SPEC,
                  _VEC_SPEC, _VEC_SPEC, _VEC_SPEC, _W_SPEC],
        out_specs=_SPLIT_SPEC, out_shape=_SPLIT_SDS,
    )(agg, hp, degp, b, g, be, w)


def _mid32_call(agg, hp, degp, b, g, be, w):
    return pl.pallas_call(
        functools.partial(_mid_body, 32), grid=(GRID,),
        in_specs=[_SPLIT_SPEC, _SPLIT_SPEC, _DEGP_SPEC,
                  _VEC_SPEC, _VEC_SPEC, _VEC_SPEC, _W64_SPEC],
        out_specs=_SPLIT32_SPEC, out_shape=_SPLIT32_SDS,
    )(agg, hp, degp, b, g, be, w)


def _final_call(agg, hp, degp, b):
    return pl.pallas_call(
        _final_body, grid=(GRID,),
        in_specs=[_SPLIT32_SPEC, _SPLIT32_SPEC, _DEGP_SPEC, _VEC64_SPEC],
        out_specs=_OUT64_SPEC, out_shape=_OUT64_SDS,
    )(agg, hp, degp, b)


def kernel(x, adj_t, W1, b1, g1, be1, W2, b2, g2, be2, W3, b3):
    src = adj_t[0].astype(jnp.int32)
    dst = adj_t[1].astype(jnp.int32)
    pad_idx = jnp.full((EP - E,), N, jnp.int32)
    src_p = jnp.concatenate([src, pad_idx])
    dst_p = jnp.concatenate([dst, pad_idx])
    src2d = src_p.reshape(EP // CH, CH)
    dst2d = dst_p.reshape(EP // CH, CH)
    srcx = jnp.concatenate([src2d, src2d + NP]).astype(jnp.int32)
    x_p = jnp.pad(x, ((0, NP - N), (0, 0)))
    ones8 = jnp.ones((CH, 8), jnp.float32)
    zeros8 = jnp.zeros((RPT, 8), jnp.float32)
    zerosH = jnp.zeros((RPT, DH), jnp.float32)
    W3p = jnp.pad(W3, ((0, 0), (0, 64 - 40)))
    b1r = b1.reshape(1, D)
    g1r = g1.reshape(1, D)
    be1r = be1.reshape(1, D)
    b2r = b2.reshape(1, D)
    g2r = g2.reshape(1, D)
    be2r = be2.reshape(1, D)
    b3r = jnp.pad(b3, (0, 64 - 40)).reshape(1, 64)
    zeros32 = jnp.zeros((RPT, 32), jnp.float32)

    degp = _deg_kernel(dst2d, ones8, zeros8).reshape(2, NP, 8)
    h1 = _prep_call(degp, x_p, W1)
    agg1 = _agg64(h1.reshape(2 * NP, DH), srcx, dst2d, zerosH)
    h2 = _mid_call(agg1.reshape(2, NP, DH), h1, degp, b1r, g1r, be1r, W2)
    agg2 = _agg64(h2.reshape(2 * NP, DH), srcx, dst2d, zerosH)
    h3 = _mid32_call(agg2.reshape(2, NP, DH), h2, degp, b2r, g2r, be2r, W3p)
    agg3 = _agg32(h3.reshape(2 * NP, 32), srcx, dst2d, zeros32)
    out = _final_call(agg3.reshape(2, NP, 32), h3, degp, b3r)
    return out[:N, :40]
